# initial kernel scaffold (unmeasured)
import jax
import jax.numpy as jnp
from jax import lax
from jax.experimental import pallas as pl
from jax.experimental.pallas import tpu as pltpu

N_DEV = 8
N_TOK = 512
D_IN = 256
D_OUT = 512
E_LOCAL = 4
CAPACITY = 12
ROWS_PER_DEV = N_TOK // N_DEV


def kernel(x, router_W, route_idx, expert_W):
    del router_W
    my = lax.axis_index("i")

    e_cols = my * E_LOCAL + jnp.arange(E_LOCAL, dtype=jnp.int32)
    oh = (route_idx == e_cols[None, :]).astype(jnp.int32)
    prior = jnp.cumsum(oh, axis=0) - oh
    keep = ((oh > 0) & (prior < CAPACITY)).astype(jnp.float32)

    def body(x_ref, keep_ref, ew_ref, out_ref, partial_ref, recv_ref,
             send_sems, recv_sems):
        my_pos = lax.axis_index("i")

        xv = x_ref[:, :]
        acc = jnp.zeros((N_TOK, D_OUT), jnp.float32)
        for j in range(E_LOCAL):
            xm = (xv * keep_ref[:, j:j + 1]).astype(jnp.bfloat16)
            w = ew_ref[j].astype(jnp.bfloat16)
            acc = acc + jnp.dot(xm, w, preferred_element_type=jnp.float32)
        partial_ref[:, :] = acc

        recv_ref[0] = partial_ref[pl.ds(my_pos * ROWS_PER_DEV, ROWS_PER_DEV), :]

        rdmas = []
        for k in range(1, N_DEV):
            tgt = lax.rem(my_pos + k, N_DEV)
            rdma = pltpu.make_async_remote_copy(
                src_ref=partial_ref.at[pl.ds(tgt * ROWS_PER_DEV, ROWS_PER_DEV), :],
                dst_ref=recv_ref.at[k],
                send_sem=send_sems.at[k],
                recv_sem=recv_sems.at[k],
                device_id=(tgt,),
                device_id_type=pl.DeviceIdType.MESH,
            )
            rdma.start()
            rdmas.append(rdma)

        for rdma in rdmas:
            rdma.wait_send()
        for rdma in rdmas:
            rdma.wait_recv()

        total = recv_ref[0]
        for k in range(1, N_DEV):
            total = total + recv_ref[k]
        out_ref[:, :] = total

    return pl.pallas_call(
        body,
        out_shape=jax.ShapeDtypeStruct((ROWS_PER_DEV, D_OUT), jnp.float32),
        in_specs=[
            pl.BlockSpec(memory_space=pltpu.VMEM),
            pl.BlockSpec(memory_space=pltpu.VMEM),
            pl.BlockSpec(memory_space=pltpu.VMEM),
        ],
        out_specs=pl.BlockSpec(memory_space=pltpu.VMEM),
        scratch_shapes=[
            pltpu.VMEM((N_TOK, D_OUT), jnp.float32),
            pltpu.VMEM((N_DEV, ROWS_PER_DEV, D_OUT), jnp.float32),
            pltpu.SemaphoreType.DMA((N_DEV,)),
            pltpu.SemaphoreType.DMA((N_DEV,)),
        ],
        compiler_params=pltpu.CompilerParams(collective_id=0),
    )(x, keep, expert_W)


# baseline (device time: 23167 ns/iter reference)
import jax
import jax.numpy as jnp
from jax import lax
from jax.experimental import pallas as pl
from jax.experimental.pallas import tpu as pltpu

N_DEV = 8
N_TOK = 512
D_IN = 256
D_OUT = 512
E_LOCAL = 4
CAPACITY = 12
ROWS_PER_DEV = N_TOK // N_DEV


def kernel(x, router_W, route_idx, expert_W):
    del router_W
    my = lax.axis_index("i")

    e_cols = my * E_LOCAL + jnp.arange(E_LOCAL, dtype=jnp.int32)
    oh = (route_idx == e_cols[None, :]).astype(jnp.int32)
    prior = jnp.cumsum(oh, axis=0) - oh
    keep = ((oh > 0) & (prior < CAPACITY)).astype(jnp.float32)

    def body(x_ref, keep_ref, ew_ref, out_ref, partial_ref, recv_ref,
             send_sems, recv_sems):
        my_pos = lax.axis_index("i")

        xv = x_ref[:, :]
        acc = jnp.zeros((N_TOK, D_OUT), jnp.float32)
        for j in range(E_LOCAL):
            xm = (xv * keep_ref[:, j:j + 1]).astype(jnp.bfloat16)
            w = ew_ref[j].astype(jnp.bfloat16)
            acc = acc + jnp.dot(xm, w, preferred_element_type=jnp.float32)
        partial_ref[:, :] = acc

        recv_ref[0] = partial_ref[pl.ds(my_pos * ROWS_PER_DEV, ROWS_PER_DEV), :]

        rdmas = []
        for k in range(1, N_DEV):
            tgt = lax.rem(my_pos + k, N_DEV)
            rdma = pltpu.make_async_remote_copy(
                src_ref=partial_ref.at[pl.ds(tgt * ROWS_PER_DEV, ROWS_PER_DEV), :],
                dst_ref=recv_ref.at[k],
                send_sem=send_sems.at[k],
                recv_sem=recv_sems.at[k],
                device_id=(tgt,),
                device_id_type=pl.DeviceIdType.MESH,
            )
            rdma.start()
            rdmas.append(rdma)

        for rdma in rdmas:
            rdma.wait_send()
        for rdma in rdmas:
            rdma.wait_recv()

        total = recv_ref[0]
        for k in range(1, N_DEV):
            total = total + recv_ref[k]
        out_ref[:, :] = total

    return pl.pallas_call(
        body,
        out_shape=jax.ShapeDtypeStruct((ROWS_PER_DEV, D_OUT), jnp.float32),
        in_specs=[
            pl.BlockSpec(memory_space=pltpu.VMEM),
            pl.BlockSpec(memory_space=pltpu.VMEM),
            pl.BlockSpec(memory_space=pltpu.VMEM),
        ],
        out_specs=pl.BlockSpec(memory_space=pltpu.VMEM),
        scratch_shapes=[
            pltpu.VMEM((N_TOK, D_OUT), jnp.float32),
            pltpu.VMEM((N_DEV, ROWS_PER_DEV, D_OUT), jnp.float32),
            pltpu.SemaphoreType.DMA((N_DEV,)),
            pltpu.SemaphoreType.DMA((N_DEV,)),
        ],
    )(x, keep, expert_W)
